# baseline (speedup 1.0000x reference)
import jax
import jax.numpy as jnp
from jax.experimental import pallas as pl
from jax.experimental.pallas import tpu as pltpu

_B, _N, _K, _CIN, _COUT = 1, 10000, 32, 128, 128
_CCAT = _CIN + 3
_NK = _N * _K
_BN = 2048
_NJ = (_N + _BN - 1) // _BN


def _body(fj4_ref, dp4_ref, wd_ref, wf_ref, b_ref,
          cat4_ref, fn_ref, dpo4_ref):
    k = pl.program_id(1)
    xk = fj4_ref[:, 0, 0, :]
    xdk = dp4_ref[:, 0, 0, :]
    cat4_ref[0:3, 0, 0, :] = xdk
    dpo4_ref[:, 0, 0, :] = xdk
    cat4_ref[3:, 0, 0, :] = xk.T
    hk = jax.lax.dot_general(
        xk, wf_ref[...], (((1,), (0,)), ((), ())),
        preferred_element_type=jnp.float32)
    hk = hk + jax.lax.dot_general(
        xdk, wd_ref[...], (((0,), (0,)), ((), ())),
        preferred_element_type=jnp.float32)

    @pl.when(k == 0)
    def _init():
        fn_ref[...] = hk

    @pl.when(k > 0)
    def _acc():
        fn_ref[...] = jnp.maximum(fn_ref[...], hk)

    @pl.when(k == _K - 1)
    def _finish():
        fn_ref[...] = jnp.maximum(fn_ref[...] + b_ref[...], 0.0)


def kernel(p, f, dp, fj, W, b):
    fj4 = jnp.transpose(fj, (0, 2, 3, 1)).reshape(_N, _K, 1, _CIN)
    dp4 = jnp.transpose(dp, (0, 1, 3, 2)).reshape(3, _K, 1, _N)
    wd = W[:, :3].T
    wf = W[:, 3:].T
    b2 = b.reshape(1, _COUT)

    cat4, fn, dpo = pl.pallas_call(
        _body,
        grid=(_NJ, _K),
        in_specs=[
            pl.BlockSpec((_BN, 1, 1, _CIN), lambda j, k: (j, k, 0, 0)),
            pl.BlockSpec((3, 1, 1, _BN), lambda j, k: (0, k, 0, j)),
            pl.BlockSpec((3, _COUT), lambda j, k: (0, 0)),
            pl.BlockSpec((_CIN, _COUT), lambda j, k: (0, 0)),
            pl.BlockSpec((1, _COUT), lambda j, k: (0, 0)),
        ],
        out_specs=[
            pl.BlockSpec((_CCAT, 1, 1, _BN), lambda j, k: (0, k, 0, j)),
            pl.BlockSpec((_BN, _COUT), lambda j, k: (j, 0)),
            pl.BlockSpec((3, 1, 1, _BN), lambda j, k: (0, k, 0, j)),
        ],
        out_shape=[
            jax.ShapeDtypeStruct((_CCAT, _K, 1, _N), jnp.float32),
            jax.ShapeDtypeStruct((_N, _COUT), jnp.float32),
            jax.ShapeDtypeStruct((3, _K, 1, _N), jnp.float32),
        ],
        compiler_params=pltpu.CompilerParams(
            dimension_semantics=("parallel", "arbitrary")),
    )(fj4, dp4, wd, wf, b2)

    fj_cat = jnp.transpose(cat4.reshape(_B, _CCAT, _K, _N), (0, 1, 3, 2))
    f_new = jnp.transpose(fn.reshape(_B, _N, _COUT), (0, 2, 1))
    dp_out = jnp.transpose(dpo.reshape(_B, 3, _K, _N), (0, 1, 3, 2))
    return (p, f_new, dp_out, fj_cat)


# speedup vs baseline: 3.8981x; 3.8981x over previous
import jax
import jax.numpy as jnp
from jax.experimental import pallas as pl
from jax.experimental.pallas import tpu as pltpu

_B, _N, _K, _CIN, _COUT = 1, 10000, 32, 128, 128
_CCAT = _CIN + 3
_NK = _N * _K
_BN = 128
_BR = _BN * _K
_NJ = (_N + _BN - 1) // _BN


def _body(fjx_ref, dp3_ref, wd_ref, wf_ref, b_ref,
          cat_ref, fn_ref, dpo_ref):
    xf = fjx_ref[...]
    xd3 = dp3_ref[...]
    cat_ref[0:3, :, :] = xd3
    dpo_ref[...] = xd3
    wf = wf_ref[...]
    wd = wd_ref[...]
    x3 = xf.reshape(_BN, _K, _CIN)
    xg = jnp.transpose(x3, (1, 0, 2))
    m = None
    for k in range(_K):
        xk = xg[k]
        cat_ref[3:, k, :] = xk.T
        hk = jax.lax.dot_general(
            xk, wf, (((1,), (0,)), ((), ())),
            preferred_element_type=jnp.float32)
        hk = hk + jax.lax.dot_general(
            xd3[:, k, :], wd, (((0,), (0,)), ((), ())),
            preferred_element_type=jnp.float32)
        m = hk if m is None else jnp.maximum(m, hk)
    fn_ref[...] = jnp.maximum(m + b_ref[...], 0.0)


def kernel(p, f, dp, fj, W, b):
    fjx = jnp.transpose(fj, (0, 2, 3, 1)).reshape(_NK, _CIN)
    dp3 = jnp.transpose(dp, (0, 1, 3, 2)).reshape(3, _K, _N)
    wd = W[:, :3].T
    wf = W[:, 3:].T
    b2 = b.reshape(1, _COUT)

    cat3, fn, dpo = pl.pallas_call(
        _body,
        grid=(_NJ,),
        in_specs=[
            pl.BlockSpec((_BR, _CIN), lambda j: (j, 0)),
            pl.BlockSpec((3, _K, _BN), lambda j: (0, 0, j)),
            pl.BlockSpec((3, _COUT), lambda j: (0, 0)),
            pl.BlockSpec((_CIN, _COUT), lambda j: (0, 0)),
            pl.BlockSpec((1, _COUT), lambda j: (0, 0)),
        ],
        out_specs=[
            pl.BlockSpec((_CCAT, _K, _BN), lambda j: (0, 0, j)),
            pl.BlockSpec((_BN, _COUT), lambda j: (j, 0)),
            pl.BlockSpec((3, _K, _BN), lambda j: (0, 0, j)),
        ],
        out_shape=[
            jax.ShapeDtypeStruct((_CCAT, _K, _N), jnp.float32),
            jax.ShapeDtypeStruct((_N, _COUT), jnp.float32),
            jax.ShapeDtypeStruct((3, _K, _N), jnp.float32),
        ],
        compiler_params=pltpu.CompilerParams(
            dimension_semantics=("parallel",)),
    )(fjx, dp3, wd, wf, b2)

    fj_cat = jnp.transpose(cat3.reshape(_B, _CCAT, _K, _N), (0, 1, 3, 2))
    f_new = jnp.transpose(fn.reshape(_B, _N, _COUT), (0, 2, 1))
    dp_out = jnp.transpose(dpo.reshape(_B, 3, _K, _N), (0, 1, 3, 2))
    return (p, f_new, dp_out, fj_cat)
